# baseline (speedup 1.0000x reference)
import functools

import jax
import jax.numpy as jnp
from jax import lax
from jax.experimental import pallas as pl
from jax.experimental.pallas import tpu as pltpu
from jax.experimental.pallas import tpu_sc as plsc

E = 8
TOP_K = 2
N = 2048
C = 1024
HS = 2048
HR = 1024
BM = 128
NBLK = N * TOP_K // BM + E
NPAD = NBLK * BM

SC_NC = 2
SC_NS = 16
NW = SC_NC * SC_NS


def _router_body(x_ref, wrx_ref, wrt_ref, temb_ref, bias_ref, idx_ref,
                 g0r_ref, g1r_ref):
    x = x_ref[...]
    logits = jnp.dot(x, wrx_ref[...], preferred_element_type=jnp.float32)
    tlog = jnp.dot(temb_ref[...], wrt_ref[...],
                   preferred_element_type=jnp.float32)
    logits = logits + tlog
    s = jax.nn.sigmoid(logits)
    cols = lax.broadcasted_iota(jnp.int32, s.shape, 1)
    valid = cols < E
    sel = s + bias_ref[...]
    big = jnp.float32(9999)
    m1 = jnp.max(sel, axis=1, keepdims=True)
    i1 = jnp.min(jnp.where((sel == m1) & valid, cols, 9999), axis=1,
                 keepdims=True)
    s1 = jnp.sum(jnp.where(cols == i1, s, 0.0), axis=1, keepdims=True)
    sel2 = jnp.where(cols == i1, -jnp.float32(3e38), sel)
    m2 = jnp.max(sel2, axis=1, keepdims=True)
    i2 = jnp.min(jnp.where((sel2 == m2) & valid, cols, 9999), axis=1,
                 keepdims=True)
    s2 = jnp.sum(jnp.where(cols == i2, s, 0.0), axis=1, keepdims=True)
    denom = s1 + s2
    ok = denom > 1e-9
    g0 = jnp.where(ok, s1 / (denom + 1e-9), 0.5)
    g1 = jnp.where(ok, s2 / (denom + 1e-9), 0.5)
    idx_ref[...] = jnp.concatenate([i1, i2], axis=1)
    g0r_ref[...] = jnp.broadcast_to(g0, (N, 16))
    g1r_ref[...] = jnp.broadcast_to(g1, (N, 16))


def _router(x_flat, t_emb, Wr, router_bias):
    wrx = jnp.pad(Wr[:C], ((0, 0), (0, 128 - E)))
    wrt = jnp.pad(Wr[C:], ((0, 0), (0, 128 - E)))
    bias = jnp.pad(router_bias, (0, 128 - E), constant_values=-1e30)
    bias = bias.reshape(1, 128)
    return pl.pallas_call(
        _router_body,
        out_shape=(
            jax.ShapeDtypeStruct((N, TOP_K), jnp.int32),
            jax.ShapeDtypeStruct((N, 16), jnp.float32),
            jax.ShapeDtypeStruct((N, 16), jnp.float32),
        ),
    )(x_flat, wrx, wrt, t_emb, bias)


def _dispatch_plan(idx2):
    ar8 = jnp.arange(E, dtype=jnp.int32)
    oh0 = (idx2[:, :1] == ar8[None, :]).astype(jnp.int32)
    oh1 = (idx2[:, 1:] == ar8[None, :]).astype(jnp.int32)
    cnt = oh0 + oh1
    incl = jnp.cumsum(cnt, axis=0)
    excl = incl - cnt
    counts = incl[-1]
    nblocks = (counts + BM - 1) // BM
    padded = nblocks * BM
    pends = jnp.cumsum(padded)
    pstarts = pends - padded
    n_active = pends[-1] // BM
    p0 = jnp.sum((pstarts[None, :] + excl) * oh0, axis=1)
    p1 = jnp.sum((pstarts[None, :] + excl) * oh1, axis=1)
    bidx = jnp.arange(NBLK, dtype=jnp.int32)
    eid = jnp.sum((bidx[:, None] * BM >= pends[None, :]).astype(jnp.int32),
                  axis=1)
    eid = jnp.minimum(eid, E - 1)
    last_eid = eid[jnp.maximum(n_active - 1, 0)]
    eid = jnp.where(bidx < n_active, eid, last_eid)
    return p0, p1, eid, n_active.reshape(1)


_TOKS_PER_W = N // NW


def _dispatch_sc(x_flat, p0_2d, p1_2d):
    mesh = plsc.VectorSubcoreMesh(core_axis_name="c", subcore_axis_name="s")

    @functools.partial(
        pl.kernel,
        out_type=jax.ShapeDtypeStruct((NPAD, C), jnp.float32),
        mesh=mesh,
        scratch_types=[
            pltpu.VMEM((_TOKS_PER_W,), jnp.int32),
            pltpu.VMEM((_TOKS_PER_W,), jnp.int32),
            pltpu.VMEM((_TOKS_PER_W, C), jnp.float32),
            pltpu.SemaphoreType.DMA,
            pltpu.SemaphoreType.DMA,
        ],
    )
    def k(x_hbm, p0_hbm, p1_hbm, out_hbm, p0_v, p1_v, xbuf, sem0, sem1):
        wid = lax.axis_index("s") * SC_NC + lax.axis_index("c")
        base = wid * _TOKS_PER_W
        pltpu.sync_copy(p0_hbm.at[wid], p0_v)
        pltpu.sync_copy(p1_hbm.at[wid], p1_v)
        pltpu.sync_copy(x_hbm.at[pl.ds(base, _TOKS_PER_W), :], xbuf)
        c0 = pltpu.async_copy(xbuf, out_hbm.at[p0_v], sem0)
        c1 = pltpu.async_copy(xbuf, out_hbm.at[p1_v], sem1)
        c0.wait()
        c1.wait()

    return k(x_flat, p0_2d, p1_2d)


def _gelu_exact(v):
    return 0.5 * v * (1.0 + lax.erf(v * (2.0 ** -0.5)))


def _grouped_body(eid_ref, nact_ref, xb_ref, w1_ref, w2_ref, y_ref):
    b = pl.program_id(0)

    @pl.when(b < nact_ref[0])
    def _():
        xb = xb_ref[...].astype(jnp.bfloat16)
        w1 = w1_ref[0].astype(jnp.bfloat16)
        h = jnp.dot(xb, w1, preferred_element_type=jnp.float32)
        h = _gelu_exact(h).astype(jnp.bfloat16)
        w2 = w2_ref[0].astype(jnp.bfloat16)
        y_ref[...] = jnp.dot(h, w2, preferred_element_type=jnp.float32)


def _grouped_matmul(x_pad, W1, W2, eid, n_active):
    grid_spec = pltpu.PrefetchScalarGridSpec(
        num_scalar_prefetch=2,
        grid=(NBLK,),
        in_specs=[
            pl.BlockSpec((BM, C), lambda b, eid, na: (b, 0)),
            pl.BlockSpec((1, C, HR), lambda b, eid, na: (eid[b], 0, 0)),
            pl.BlockSpec((1, HR, C), lambda b, eid, na: (eid[b], 0, 0)),
        ],
        out_specs=pl.BlockSpec((BM, C), lambda b, eid, na: (b, 0)),
    )
    return pl.pallas_call(
        _grouped_body,
        grid_spec=grid_spec,
        out_shape=jax.ShapeDtypeStruct((NPAD, C), jnp.float32),
    )(eid, n_active, x_pad, W1, W2)


_HSC = 512


def _shared_body(x_ref, w1_ref, w3_ref, w2_ref, o_ref):
    k = pl.program_id(0)
    xb = x_ref[...].astype(jnp.bfloat16)
    a = jnp.dot(xb, w1_ref[...].astype(jnp.bfloat16),
                preferred_element_type=jnp.float32)
    g = jnp.dot(xb, w3_ref[...].astype(jnp.bfloat16),
                preferred_element_type=jnp.float32)
    h = ((a * jax.nn.sigmoid(a)) * g).astype(jnp.bfloat16)
    part = jnp.dot(h, w2_ref[...].astype(jnp.bfloat16),
                   preferred_element_type=jnp.float32)

    @pl.when(k == 0)
    def _():
        o_ref[...] = part

    @pl.when(k > 0)
    def _():
        o_ref[...] += part


def _shared(x_flat, Ws1, Ws3, Ws2):
    return pl.pallas_call(
        _shared_body,
        grid=(HS // _HSC,),
        in_specs=[
            pl.BlockSpec((N, C), lambda k: (0, 0)),
            pl.BlockSpec((C, _HSC), lambda k: (0, k)),
            pl.BlockSpec((C, _HSC), lambda k: (0, k)),
            pl.BlockSpec((_HSC, C), lambda k: (k, 0)),
        ],
        out_specs=pl.BlockSpec((N, C), lambda k: (0, 0)),
        out_shape=jax.ShapeDtypeStruct((N, C), jnp.float32),
    )(x_flat, Ws1, Ws3, Ws2)


_TOK_PER_W = N // NW
_CCHUNK = 16


def _combine_sc(shared, y, p0, p1, g0r, g1r):
    mesh = plsc.VectorSubcoreMesh(core_axis_name="c", subcore_axis_name="s")
    inv3 = jnp.float32(1.0 / (1 + TOP_K))

    nchunks = _TOK_PER_W // _CCHUNK

    @functools.partial(
        pl.kernel,
        out_type=jax.ShapeDtypeStruct((N, C), jnp.float32),
        mesh=mesh,
        scratch_types=[
            pltpu.VMEM((_TOK_PER_W,), jnp.int32),
            pltpu.VMEM((_TOK_PER_W,), jnp.int32),
            pltpu.VMEM((_TOK_PER_W, 16), jnp.float32),
            pltpu.VMEM((_TOK_PER_W, 16), jnp.float32),
            pltpu.VMEM((2, _CCHUNK, C), jnp.float32),
            pltpu.VMEM((2, _CCHUNK, C), jnp.float32),
            pltpu.VMEM((_CCHUNK, C), jnp.float32),
            pltpu.VMEM((_CCHUNK, C), jnp.float32),
            pltpu.SemaphoreType.DMA,
            pltpu.SemaphoreType.DMA,
            pltpu.SemaphoreType.DMA,
        ],
    )
    def k(sh_hbm, y_hbm, p0_hbm, p1_hbm, g0_hbm, g1_hbm, out_hbm,
          p0_v, p1_v, g0_v, g1_v, y0b, y1b, shb, ob, sem0, sem1, osem):
        wid = lax.axis_index("s") * SC_NC + lax.axis_index("c")
        base = wid * _TOK_PER_W
        pltpu.sync_copy(p0_hbm.at[pl.ds(base, _TOK_PER_W)], p0_v)
        pltpu.sync_copy(p1_hbm.at[pl.ds(base, _TOK_PER_W)], p1_v)
        pltpu.sync_copy(g0_hbm.at[pl.ds(base, _TOK_PER_W), :], g0_v)
        pltpu.sync_copy(g1_hbm.at[pl.ds(base, _TOK_PER_W), :], g1_v)
        sems = (sem0, sem1)

        def gather(cc):
            sl = cc % 2
            c0 = pltpu.async_copy(
                y_hbm.at[p0_v.at[pl.ds(cc * _CCHUNK, _CCHUNK)]], y0b.at[sl],
                sems[sl])
            c1 = pltpu.async_copy(
                y_hbm.at[p1_v.at[pl.ds(cc * _CCHUNK, _CCHUNK)]], y1b.at[sl],
                sems[sl])
            return c0, c1

        pend = gather(0)
        owait = None
        for cc in range(nchunks):
            sl = cc % 2
            nxt = gather(cc + 1) if cc + 1 < nchunks else None
            pltpu.sync_copy(sh_hbm.at[pl.ds(base + cc * _CCHUNK, _CCHUNK), :],
                            shb)
            pend[0].wait()
            pend[1].wait()
            if owait is not None:
                owait.wait()

            def tok(t, _):
                g0vec = g0_v[cc * _CCHUNK + t, :]
                g1vec = g1_v[cc * _CCHUNK + t, :]
                for ch in range(C // 16):
                    s2 = pl.ds(ch * 16, 16)
                    ob[t, s2] = (shb[t, s2] + g0vec * y0b[sl, t, s2]
                                 + g1vec * y1b[sl, t, s2]) * inv3
                return 0

            lax.fori_loop(0, _CCHUNK, tok, 0)
            owait = pltpu.async_copy(
                ob, out_hbm.at[pl.ds(base + cc * _CCHUNK, _CCHUNK), :],
                osem)
            pend = nxt
        owait.wait()

    return k(shared, y, p0, p1, g0r, g1r)


def kernel(x, t_emb, Ws1, Ws3, Ws2, W1, W2, Wr, router_bias):
    B, T, Cc = x.shape
    x_flat = x.reshape(-1, Cc)
    idx2, g0r, g1r = _router(x_flat, t_emb, Wr, router_bias)
    p0, p1, eid, n_active = _dispatch_plan(idx2)
    x_pad = _dispatch_sc(x_flat, p0.reshape(NW, _TOKS_PER_W),
                         p1.reshape(NW, _TOKS_PER_W))
    y = _grouped_matmul(x_pad, W1, W2, eid, n_active)
    sh = _shared(x_flat, Ws1, Ws3, Ws2)
    out = _combine_sc(sh, y, p0, p1, g0r, g1r)
    return out.reshape(B, T, Cc)


# speedup vs baseline: 1.0441x; 1.0441x over previous
import functools

import jax
import jax.numpy as jnp
from jax import lax
from jax.experimental import pallas as pl
from jax.experimental.pallas import tpu as pltpu
from jax.experimental.pallas import tpu_sc as plsc

E = 8
TOP_K = 2
N = 2048
C = 1024
HS = 2048
HR = 1024
BM = 256
NBLK = N * TOP_K // BM + E
NPAD = NBLK * BM

SC_NC = 2
SC_NS = 16
NW = SC_NC * SC_NS


def _router_body(x_ref, wrx_ref, wrt_ref, temb_ref, bias_ref, idx_ref,
                 g0r_ref, g1r_ref):
    x = x_ref[...]
    logits = jnp.dot(x, wrx_ref[...], preferred_element_type=jnp.float32)
    tlog = jnp.dot(temb_ref[...], wrt_ref[...],
                   preferred_element_type=jnp.float32)
    logits = logits + tlog
    s = jax.nn.sigmoid(logits)
    cols = lax.broadcasted_iota(jnp.int32, s.shape, 1)
    valid = cols < E
    sel = s + bias_ref[...]
    big = jnp.float32(9999)
    m1 = jnp.max(sel, axis=1, keepdims=True)
    i1 = jnp.min(jnp.where((sel == m1) & valid, cols, 9999), axis=1,
                 keepdims=True)
    s1 = jnp.sum(jnp.where(cols == i1, s, 0.0), axis=1, keepdims=True)
    sel2 = jnp.where(cols == i1, -jnp.float32(3e38), sel)
    m2 = jnp.max(sel2, axis=1, keepdims=True)
    i2 = jnp.min(jnp.where((sel2 == m2) & valid, cols, 9999), axis=1,
                 keepdims=True)
    s2 = jnp.sum(jnp.where(cols == i2, s, 0.0), axis=1, keepdims=True)
    denom = s1 + s2
    ok = denom > 1e-9
    g0 = jnp.where(ok, s1 / (denom + 1e-9), 0.5)
    g1 = jnp.where(ok, s2 / (denom + 1e-9), 0.5)
    idx_ref[...] = jnp.concatenate([i1, i2], axis=1)
    g0r_ref[...] = jnp.broadcast_to(g0, (N, 16))
    g1r_ref[...] = jnp.broadcast_to(g1, (N, 16))


def _router(x_flat, t_emb, Wr, router_bias):
    wrx = jnp.pad(Wr[:C], ((0, 0), (0, 128 - E)))
    wrt = jnp.pad(Wr[C:], ((0, 0), (0, 128 - E)))
    bias = jnp.pad(router_bias, (0, 128 - E), constant_values=-1e30)
    bias = bias.reshape(1, 128)
    return pl.pallas_call(
        _router_body,
        out_shape=(
            jax.ShapeDtypeStruct((N, TOP_K), jnp.int32),
            jax.ShapeDtypeStruct((N, 16), jnp.float32),
            jax.ShapeDtypeStruct((N, 16), jnp.float32),
        ),
    )(x_flat, wrx, wrt, t_emb, bias)


def _dispatch_plan(idx2):
    ar8 = jnp.arange(E, dtype=jnp.int32)
    oh0 = (idx2[:, :1] == ar8[None, :]).astype(jnp.int32)
    oh1 = (idx2[:, 1:] == ar8[None, :]).astype(jnp.int32)
    cnt = oh0 + oh1
    incl = jnp.cumsum(cnt, axis=0)
    excl = incl - cnt
    counts = incl[-1]
    nblocks = (counts + BM - 1) // BM
    padded = nblocks * BM
    pends = jnp.cumsum(padded)
    pstarts = pends - padded
    n_active = pends[-1] // BM
    p0 = jnp.sum((pstarts[None, :] + excl) * oh0, axis=1)
    p1 = jnp.sum((pstarts[None, :] + excl) * oh1, axis=1)
    bidx = jnp.arange(NBLK, dtype=jnp.int32)
    eid = jnp.sum((bidx[:, None] * BM >= pends[None, :]).astype(jnp.int32),
                  axis=1)
    eid = jnp.minimum(eid, E - 1)
    last_eid = eid[jnp.maximum(n_active - 1, 0)]
    eid = jnp.where(bidx < n_active, eid, last_eid)
    return p0, p1, eid, n_active.reshape(1)


_TOKS_PER_W = N // NW


def _dispatch_sc(x_flat, p0_2d, p1_2d):
    mesh = plsc.VectorSubcoreMesh(core_axis_name="c", subcore_axis_name="s")

    @functools.partial(
        pl.kernel,
        out_type=jax.ShapeDtypeStruct((NPAD, C), jnp.float32),
        mesh=mesh,
        scratch_types=[
            pltpu.VMEM((_TOKS_PER_W,), jnp.int32),
            pltpu.VMEM((_TOKS_PER_W,), jnp.int32),
            pltpu.VMEM((_TOKS_PER_W, C), jnp.float32),
            pltpu.SemaphoreType.DMA,
            pltpu.SemaphoreType.DMA,
        ],
    )
    def k(x_hbm, p0_hbm, p1_hbm, out_hbm, p0_v, p1_v, xbuf, sem0, sem1):
        wid = lax.axis_index("s") * SC_NC + lax.axis_index("c")
        base = wid * _TOKS_PER_W
        pltpu.sync_copy(p0_hbm.at[wid], p0_v)
        pltpu.sync_copy(p1_hbm.at[wid], p1_v)
        pltpu.sync_copy(x_hbm.at[pl.ds(base, _TOKS_PER_W), :], xbuf)
        c0 = pltpu.async_copy(xbuf, out_hbm.at[p0_v], sem0)
        c1 = pltpu.async_copy(xbuf, out_hbm.at[p1_v], sem1)
        c0.wait()
        c1.wait()

    return k(x_flat, p0_2d, p1_2d)


def _gelu_exact(v):
    return 0.5 * v * (1.0 + lax.erf(v * (2.0 ** -0.5)))


def _grouped_body(eid_ref, nact_ref, xb_ref, w1_ref, w2_ref, y_ref):
    b = pl.program_id(0)

    @pl.when(b < nact_ref[0])
    def _():
        xb = xb_ref[...].astype(jnp.bfloat16)
        w1 = w1_ref[0].astype(jnp.bfloat16)
        h = jnp.dot(xb, w1, preferred_element_type=jnp.float32)
        h = _gelu_exact(h).astype(jnp.bfloat16)
        w2 = w2_ref[0].astype(jnp.bfloat16)
        y_ref[...] = jnp.dot(h, w2, preferred_element_type=jnp.float32)


def _grouped_matmul(x_pad, W1, W2, eid, n_active):
    grid_spec = pltpu.PrefetchScalarGridSpec(
        num_scalar_prefetch=2,
        grid=(NBLK,),
        in_specs=[
            pl.BlockSpec((BM, C), lambda b, eid, na: (b, 0)),
            pl.BlockSpec((1, C, HR), lambda b, eid, na: (eid[b], 0, 0)),
            pl.BlockSpec((1, HR, C), lambda b, eid, na: (eid[b], 0, 0)),
        ],
        out_specs=pl.BlockSpec((BM, C), lambda b, eid, na: (b, 0)),
    )
    return pl.pallas_call(
        _grouped_body,
        grid_spec=grid_spec,
        out_shape=jax.ShapeDtypeStruct((NPAD, C), jnp.float32),
    )(eid, n_active, x_pad, W1, W2)


_HSC = 512


def _shared_body(x_ref, w1_ref, w3_ref, w2_ref, o_ref):
    k = pl.program_id(0)
    xb = x_ref[...].astype(jnp.bfloat16)
    a = jnp.dot(xb, w1_ref[...].astype(jnp.bfloat16),
                preferred_element_type=jnp.float32)
    g = jnp.dot(xb, w3_ref[...].astype(jnp.bfloat16),
                preferred_element_type=jnp.float32)
    h = ((a * jax.nn.sigmoid(a)) * g).astype(jnp.bfloat16)
    part = jnp.dot(h, w2_ref[...].astype(jnp.bfloat16),
                   preferred_element_type=jnp.float32)

    @pl.when(k == 0)
    def _():
        o_ref[...] = part

    @pl.when(k > 0)
    def _():
        o_ref[...] += part


def _shared(x_flat, Ws1, Ws3, Ws2):
    return pl.pallas_call(
        _shared_body,
        grid=(HS // _HSC,),
        in_specs=[
            pl.BlockSpec((N, C), lambda k: (0, 0)),
            pl.BlockSpec((C, _HSC), lambda k: (0, k)),
            pl.BlockSpec((C, _HSC), lambda k: (0, k)),
            pl.BlockSpec((_HSC, C), lambda k: (k, 0)),
        ],
        out_specs=pl.BlockSpec((N, C), lambda k: (0, 0)),
        out_shape=jax.ShapeDtypeStruct((N, C), jnp.float32),
    )(x_flat, Ws1, Ws3, Ws2)


_TOK_PER_W = N // NW
_CCHUNK = 16


def _combine_sc(shared, y, p0, p1, g0r, g1r):
    mesh = plsc.VectorSubcoreMesh(core_axis_name="c", subcore_axis_name="s")
    inv3 = jnp.float32(1.0 / (1 + TOP_K))

    nchunks = _TOK_PER_W // _CCHUNK

    @functools.partial(
        pl.kernel,
        out_type=jax.ShapeDtypeStruct((N, C), jnp.float32),
        mesh=mesh,
        scratch_types=[
            pltpu.VMEM((_TOK_PER_W,), jnp.int32),
            pltpu.VMEM((_TOK_PER_W,), jnp.int32),
            pltpu.VMEM((_TOK_PER_W, 16), jnp.float32),
            pltpu.VMEM((_TOK_PER_W, 16), jnp.float32),
            pltpu.VMEM((2, _CCHUNK, C), jnp.float32),
            pltpu.VMEM((2, _CCHUNK, C), jnp.float32),
            pltpu.VMEM((_CCHUNK, C), jnp.float32),
            pltpu.VMEM((_CCHUNK, C), jnp.float32),
            pltpu.SemaphoreType.DMA,
            pltpu.SemaphoreType.DMA,
            pltpu.SemaphoreType.DMA,
        ],
    )
    def k(sh_hbm, y_hbm, p0_hbm, p1_hbm, g0_hbm, g1_hbm, out_hbm,
          p0_v, p1_v, g0_v, g1_v, y0b, y1b, shb, ob, sem0, sem1, osem):
        wid = lax.axis_index("s") * SC_NC + lax.axis_index("c")
        base = wid * _TOK_PER_W
        pltpu.sync_copy(p0_hbm.at[pl.ds(base, _TOK_PER_W)], p0_v)
        pltpu.sync_copy(p1_hbm.at[pl.ds(base, _TOK_PER_W)], p1_v)
        pltpu.sync_copy(g0_hbm.at[pl.ds(base, _TOK_PER_W), :], g0_v)
        pltpu.sync_copy(g1_hbm.at[pl.ds(base, _TOK_PER_W), :], g1_v)
        sems = (sem0, sem1)

        def gather(cc):
            sl = cc % 2
            c0 = pltpu.async_copy(
                y_hbm.at[p0_v.at[pl.ds(cc * _CCHUNK, _CCHUNK)]], y0b.at[sl],
                sems[sl])
            c1 = pltpu.async_copy(
                y_hbm.at[p1_v.at[pl.ds(cc * _CCHUNK, _CCHUNK)]], y1b.at[sl],
                sems[sl])
            return c0, c1

        pend = gather(0)
        owait = None
        for cc in range(nchunks):
            sl = cc % 2
            nxt = gather(cc + 1) if cc + 1 < nchunks else None
            pltpu.sync_copy(sh_hbm.at[pl.ds(base + cc * _CCHUNK, _CCHUNK), :],
                            shb)
            pend[0].wait()
            pend[1].wait()
            if owait is not None:
                owait.wait()

            def tok(t, _):
                g0vec = g0_v[cc * _CCHUNK + t, :]
                g1vec = g1_v[cc * _CCHUNK + t, :]
                for ch in range(C // 16):
                    s2 = pl.ds(ch * 16, 16)
                    ob[t, s2] = (shb[t, s2] + g0vec * y0b[sl, t, s2]
                                 + g1vec * y1b[sl, t, s2]) * inv3
                return 0

            lax.fori_loop(0, _CCHUNK, tok, 0)
            owait = pltpu.async_copy(
                ob, out_hbm.at[pl.ds(base + cc * _CCHUNK, _CCHUNK), :],
                osem)
            pend = nxt
        owait.wait()

    return k(shared, y, p0, p1, g0r, g1r)


def kernel(x, t_emb, Ws1, Ws3, Ws2, W1, W2, Wr, router_bias):
    B, T, Cc = x.shape
    x_flat = x.reshape(-1, Cc)
    idx2, g0r, g1r = _router(x_flat, t_emb, Wr, router_bias)
    p0, p1, eid, n_active = _dispatch_plan(idx2)
    x_pad = _dispatch_sc(x_flat, p0.reshape(NW, _TOKS_PER_W),
                         p1.reshape(NW, _TOKS_PER_W))
    y = _grouped_matmul(x_pad, W1, W2, eid, n_active)
    sh = _shared(x_flat, Ws1, Ws3, Ws2)
    out = _combine_sc(sh, y, p0, p1, g0r, g1r)
    return out.reshape(B, T, Cc)


# speedup vs baseline: 1.0755x; 1.0301x over previous
import functools

import jax
import jax.numpy as jnp
from jax import lax
from jax.experimental import pallas as pl
from jax.experimental.pallas import tpu as pltpu
from jax.experimental.pallas import tpu_sc as plsc

E = 8
TOP_K = 2
N = 2048
C = 1024
HS = 2048
HR = 1024
BM = 256
NBLK = N * TOP_K // BM + E
NPAD = NBLK * BM

SC_NC = 2
SC_NS = 16
NW = SC_NC * SC_NS


def _router_body(x_ref, wrx_ref, wrt_ref, temb_ref, bias_ref, idx_ref,
                 g0r_ref, g1r_ref):
    x = x_ref[...]
    logits = jnp.dot(x, wrx_ref[...], preferred_element_type=jnp.float32)
    tlog = jnp.dot(temb_ref[...], wrt_ref[...],
                   preferred_element_type=jnp.float32)
    logits = logits + tlog
    s = jax.nn.sigmoid(logits)
    cols = lax.broadcasted_iota(jnp.int32, s.shape, 1)
    valid = cols < E
    sel = s + bias_ref[...]
    big = jnp.float32(9999)
    m1 = jnp.max(sel, axis=1, keepdims=True)
    i1 = jnp.min(jnp.where((sel == m1) & valid, cols, 9999), axis=1,
                 keepdims=True)
    s1 = jnp.sum(jnp.where(cols == i1, s, 0.0), axis=1, keepdims=True)
    sel2 = jnp.where(cols == i1, -jnp.float32(3e38), sel)
    m2 = jnp.max(sel2, axis=1, keepdims=True)
    i2 = jnp.min(jnp.where((sel2 == m2) & valid, cols, 9999), axis=1,
                 keepdims=True)
    s2 = jnp.sum(jnp.where(cols == i2, s, 0.0), axis=1, keepdims=True)
    denom = s1 + s2
    ok = denom > 1e-9
    g0 = jnp.where(ok, s1 / (denom + 1e-9), 0.5)
    g1 = jnp.where(ok, s2 / (denom + 1e-9), 0.5)
    idx_ref[...] = jnp.concatenate([i1, i2], axis=1)
    g0r_ref[...] = jnp.broadcast_to(g0, (N, 16))
    g1r_ref[...] = jnp.broadcast_to(g1, (N, 16))


def _router(x_flat, t_emb, Wr, router_bias):
    wrx = jnp.pad(Wr[:C], ((0, 0), (0, 128 - E)))
    wrt = jnp.pad(Wr[C:], ((0, 0), (0, 128 - E)))
    bias = jnp.pad(router_bias, (0, 128 - E), constant_values=-1e30)
    bias = bias.reshape(1, 128)
    return pl.pallas_call(
        _router_body,
        out_shape=(
            jax.ShapeDtypeStruct((N, TOP_K), jnp.int32),
            jax.ShapeDtypeStruct((N, 16), jnp.float32),
            jax.ShapeDtypeStruct((N, 16), jnp.float32),
        ),
    )(x_flat, wrx, wrt, t_emb, bias)


def _dispatch_plan(idx2):
    ar8 = jnp.arange(E, dtype=jnp.int32)
    oh0 = (idx2[:, :1] == ar8[None, :]).astype(jnp.int32)
    oh1 = (idx2[:, 1:] == ar8[None, :]).astype(jnp.int32)
    cnt = oh0 + oh1
    incl = jnp.cumsum(cnt, axis=0)
    excl = incl - cnt
    counts = incl[-1]
    nblocks = (counts + BM - 1) // BM
    padded = nblocks * BM
    pends = jnp.cumsum(padded)
    pstarts = pends - padded
    n_active = pends[-1] // BM
    p0 = jnp.sum((pstarts[None, :] + excl) * oh0, axis=1)
    p1 = jnp.sum((pstarts[None, :] + excl) * oh1, axis=1)
    bidx = jnp.arange(NBLK, dtype=jnp.int32)
    eid = jnp.sum((bidx[:, None] * BM >= pends[None, :]).astype(jnp.int32),
                  axis=1)
    eid = jnp.minimum(eid, E - 1)
    last_eid = eid[jnp.maximum(n_active - 1, 0)]
    eid = jnp.where(bidx < n_active, eid, last_eid)
    return p0, p1, eid, n_active.reshape(1)


_TOKS_PER_W = N // NW


def _dispatch_sc(x_flat, p0_2d, p1_2d):
    mesh = plsc.VectorSubcoreMesh(core_axis_name="c", subcore_axis_name="s")

    @functools.partial(
        pl.kernel,
        out_type=jax.ShapeDtypeStruct((NPAD, C), jnp.float32),
        mesh=mesh,
        scratch_types=[
            pltpu.VMEM((_TOKS_PER_W,), jnp.int32),
            pltpu.VMEM((_TOKS_PER_W,), jnp.int32),
            pltpu.VMEM((_TOKS_PER_W, C), jnp.float32),
            pltpu.SemaphoreType.DMA,
            pltpu.SemaphoreType.DMA,
        ],
    )
    def k(x_hbm, p0_hbm, p1_hbm, out_hbm, p0_v, p1_v, xbuf, sem0, sem1):
        wid = lax.axis_index("s") * SC_NC + lax.axis_index("c")
        base = wid * _TOKS_PER_W
        pltpu.sync_copy(p0_hbm.at[wid], p0_v)
        pltpu.sync_copy(p1_hbm.at[wid], p1_v)
        pltpu.sync_copy(x_hbm.at[pl.ds(base, _TOKS_PER_W), :], xbuf)
        c0 = pltpu.async_copy(xbuf, out_hbm.at[p0_v], sem0)
        c1 = pltpu.async_copy(xbuf, out_hbm.at[p1_v], sem1)
        c0.wait()
        c1.wait()

    return k(x_flat, p0_2d, p1_2d)


def _gelu_exact(v):
    return 0.5 * v * (1.0 + lax.erf(v * (2.0 ** -0.5)))


def _grouped_body(eid_ref, nact_ref, xb_ref, w1_ref, w2_ref, y_ref):
    b = pl.program_id(0)

    @pl.when(b < nact_ref[0])
    def _():
        xb = xb_ref[...].astype(jnp.bfloat16)
        w1 = w1_ref[0].astype(jnp.bfloat16)
        h = jnp.dot(xb, w1, preferred_element_type=jnp.float32)
        h = _gelu_exact(h).astype(jnp.bfloat16)
        w2 = w2_ref[0].astype(jnp.bfloat16)
        y_ref[...] = jnp.dot(h, w2, preferred_element_type=jnp.float32)


def _grouped_matmul(x_pad, W1, W2, eid, n_active):
    grid_spec = pltpu.PrefetchScalarGridSpec(
        num_scalar_prefetch=2,
        grid=(NBLK,),
        in_specs=[
            pl.BlockSpec((BM, C),
                         lambda b, eid, na: (jnp.minimum(b, na[0] - 1), 0)),
            pl.BlockSpec((1, C, HR), lambda b, eid, na: (eid[b], 0, 0)),
            pl.BlockSpec((1, HR, C), lambda b, eid, na: (eid[b], 0, 0)),
        ],
        out_specs=pl.BlockSpec(
            (BM, C), lambda b, eid, na: (jnp.minimum(b, na[0] - 1), 0)),
    )
    return pl.pallas_call(
        _grouped_body,
        grid_spec=grid_spec,
        out_shape=jax.ShapeDtypeStruct((NPAD, C), jnp.float32),
    )(eid, n_active, x_pad, W1, W2)


def _shared_body(x_ref, w1_ref, w3_ref, w2_ref, o_ref):
    xb = x_ref[...].astype(jnp.bfloat16)
    a = jnp.dot(xb, w1_ref[...].astype(jnp.bfloat16),
                preferred_element_type=jnp.float32)
    g = jnp.dot(xb, w3_ref[...].astype(jnp.bfloat16),
                preferred_element_type=jnp.float32)
    h = ((a * jax.nn.sigmoid(a)) * g).astype(jnp.bfloat16)
    o_ref[...] = jnp.dot(h, w2_ref[...].astype(jnp.bfloat16),
                         preferred_element_type=jnp.float32)


def _shared(x_flat, Ws1, Ws3, Ws2):
    nb = N // 256
    return pl.pallas_call(
        _shared_body,
        grid=(nb,),
        in_specs=[
            pl.BlockSpec((256, C), lambda b: (b, 0)),
            pl.BlockSpec((C, HS), lambda b: (0, 0)),
            pl.BlockSpec((C, HS), lambda b: (0, 0)),
            pl.BlockSpec((HS, C), lambda b: (0, 0)),
        ],
        out_specs=pl.BlockSpec((256, C), lambda b: (b, 0)),
        out_shape=jax.ShapeDtypeStruct((N, C), jnp.float32),
    )(x_flat, Ws1, Ws3, Ws2)


_TOK_PER_W = N // NW
_CCHUNK = 16


def _combine_sc(shared, y, p0, p1, g0r, g1r):
    mesh = plsc.VectorSubcoreMesh(core_axis_name="c", subcore_axis_name="s")
    inv3 = jnp.float32(1.0 / (1 + TOP_K))

    nchunks = _TOK_PER_W // _CCHUNK

    @functools.partial(
        pl.kernel,
        out_type=jax.ShapeDtypeStruct((N, C), jnp.float32),
        mesh=mesh,
        scratch_types=[
            pltpu.VMEM((_TOK_PER_W,), jnp.int32),
            pltpu.VMEM((_TOK_PER_W,), jnp.int32),
            pltpu.VMEM((_TOK_PER_W, 16), jnp.float32),
            pltpu.VMEM((_TOK_PER_W, 16), jnp.float32),
            pltpu.VMEM((2, _CCHUNK, C), jnp.float32),
            pltpu.VMEM((2, _CCHUNK, C), jnp.float32),
            pltpu.VMEM((_CCHUNK, C), jnp.float32),
            pltpu.VMEM((_CCHUNK, C), jnp.float32),
            pltpu.SemaphoreType.DMA,
            pltpu.SemaphoreType.DMA,
            pltpu.SemaphoreType.DMA,
        ],
    )
    def k(sh_hbm, y_hbm, p0_hbm, p1_hbm, g0_hbm, g1_hbm, out_hbm,
          p0_v, p1_v, g0_v, g1_v, y0b, y1b, shb, ob, sem0, sem1, osem):
        wid = lax.axis_index("s") * SC_NC + lax.axis_index("c")
        base = wid * _TOK_PER_W
        pltpu.sync_copy(p0_hbm.at[pl.ds(base, _TOK_PER_W)], p0_v)
        pltpu.sync_copy(p1_hbm.at[pl.ds(base, _TOK_PER_W)], p1_v)
        pltpu.sync_copy(g0_hbm.at[pl.ds(base, _TOK_PER_W), :], g0_v)
        pltpu.sync_copy(g1_hbm.at[pl.ds(base, _TOK_PER_W), :], g1_v)
        sems = (sem0, sem1)

        def gather(cc):
            sl = cc % 2
            c0 = pltpu.async_copy(
                y_hbm.at[p0_v.at[pl.ds(cc * _CCHUNK, _CCHUNK)]], y0b.at[sl],
                sems[sl])
            c1 = pltpu.async_copy(
                y_hbm.at[p1_v.at[pl.ds(cc * _CCHUNK, _CCHUNK)]], y1b.at[sl],
                sems[sl])
            return c0, c1

        pend = gather(0)
        owait = None
        for cc in range(nchunks):
            sl = cc % 2
            nxt = gather(cc + 1) if cc + 1 < nchunks else None
            pltpu.sync_copy(sh_hbm.at[pl.ds(base + cc * _CCHUNK, _CCHUNK), :],
                            shb)
            pend[0].wait()
            pend[1].wait()
            if owait is not None:
                owait.wait()

            def tok(t, _):
                g0vec = g0_v[cc * _CCHUNK + t, :]
                g1vec = g1_v[cc * _CCHUNK + t, :]
                for ch in range(C // 16):
                    s2 = pl.ds(ch * 16, 16)
                    ob[t, s2] = (shb[t, s2] + g0vec * y0b[sl, t, s2]
                                 + g1vec * y1b[sl, t, s2]) * inv3
                return 0

            lax.fori_loop(0, _CCHUNK, tok, 0)
            owait = pltpu.async_copy(
                ob, out_hbm.at[pl.ds(base + cc * _CCHUNK, _CCHUNK), :],
                osem)
            pend = nxt
        owait.wait()

    return k(shared, y, p0, p1, g0r, g1r)


def kernel(x, t_emb, Ws1, Ws3, Ws2, W1, W2, Wr, router_bias):
    B, T, Cc = x.shape
    x_flat = x.reshape(-1, Cc)
    idx2, g0r, g1r = _router(x_flat, t_emb, Wr, router_bias)
    p0, p1, eid, n_active = _dispatch_plan(idx2)
    x_pad = _dispatch_sc(x_flat, p0.reshape(NW, _TOKS_PER_W),
                         p1.reshape(NW, _TOKS_PER_W))
    y = _grouped_matmul(x_pad, W1, W2, eid, n_active)
    sh = _shared(x_flat, Ws1, Ws3, Ws2)
    out = _combine_sc(sh, y, p0, p1, g0r, g1r)
    return out.reshape(B, T, Cc)


# speedup vs baseline: 1.1216x; 1.0428x over previous
import functools

import jax
import jax.numpy as jnp
from jax import lax
from jax.experimental import pallas as pl
from jax.experimental.pallas import tpu as pltpu
from jax.experimental.pallas import tpu_sc as plsc

E = 8
TOP_K = 2
N = 2048
C = 1024
HS = 2048
HR = 1024
BM = 256
NBLK = N * TOP_K // BM + E
NPAD = NBLK * BM

SC_NC = 2
SC_NS = 16
NW = SC_NC * SC_NS


def _router_body(x_ref, wrx_ref, wrt_ref, temb_ref, bias_ref, idx_ref,
                 g0r_ref, g1r_ref):
    x = x_ref[...]
    logits = jnp.dot(x, wrx_ref[...], preferred_element_type=jnp.float32)
    tlog = jnp.dot(temb_ref[...], wrt_ref[...],
                   preferred_element_type=jnp.float32)
    logits = logits + tlog
    s = jax.nn.sigmoid(logits)
    cols = lax.broadcasted_iota(jnp.int32, s.shape, 1)
    valid = cols < E
    sel = s + bias_ref[...]
    big = jnp.float32(9999)
    m1 = jnp.max(sel, axis=1, keepdims=True)
    i1 = jnp.min(jnp.where((sel == m1) & valid, cols, 9999), axis=1,
                 keepdims=True)
    s1 = jnp.sum(jnp.where(cols == i1, s, 0.0), axis=1, keepdims=True)
    sel2 = jnp.where(cols == i1, -jnp.float32(3e38), sel)
    m2 = jnp.max(sel2, axis=1, keepdims=True)
    i2 = jnp.min(jnp.where((sel2 == m2) & valid, cols, 9999), axis=1,
                 keepdims=True)
    s2 = jnp.sum(jnp.where(cols == i2, s, 0.0), axis=1, keepdims=True)
    denom = s1 + s2
    ok = denom > 1e-9
    g0 = jnp.where(ok, s1 / (denom + 1e-9), 0.5)
    g1 = jnp.where(ok, s2 / (denom + 1e-9), 0.5)
    idx_ref[...] = jnp.concatenate([i1, i2], axis=1)
    g0r_ref[...] = jnp.broadcast_to(g0, (N, 16))
    g1r_ref[...] = jnp.broadcast_to(g1, (N, 16))


def _router(x_flat, t_emb, Wr, router_bias):
    wrx = jnp.pad(Wr[:C], ((0, 0), (0, 128 - E)))
    wrt = jnp.pad(Wr[C:], ((0, 0), (0, 128 - E)))
    bias = jnp.pad(router_bias, (0, 128 - E), constant_values=-1e30)
    bias = bias.reshape(1, 128)
    return pl.pallas_call(
        _router_body,
        out_shape=(
            jax.ShapeDtypeStruct((N, TOP_K), jnp.int32),
            jax.ShapeDtypeStruct((N, 16), jnp.float32),
            jax.ShapeDtypeStruct((N, 16), jnp.float32),
        ),
    )(x_flat, wrx, wrt, t_emb, bias)


def _dispatch_plan(idx2):
    ar8 = jnp.arange(E, dtype=jnp.int32)
    oh0 = (idx2[:, :1] == ar8[None, :]).astype(jnp.int32)
    oh1 = (idx2[:, 1:] == ar8[None, :]).astype(jnp.int32)
    cnt = oh0 + oh1
    incl = jnp.cumsum(cnt, axis=0)
    excl = incl - cnt
    counts = incl[-1]
    nblocks = (counts + BM - 1) // BM
    padded = nblocks * BM
    pends = jnp.cumsum(padded)
    pstarts = pends - padded
    n_active = pends[-1] // BM
    p0 = jnp.sum((pstarts[None, :] + excl) * oh0, axis=1)
    p1 = jnp.sum((pstarts[None, :] + excl) * oh1, axis=1)
    bidx = jnp.arange(NBLK, dtype=jnp.int32)
    eid = jnp.sum((bidx[:, None] * BM >= pends[None, :]).astype(jnp.int32),
                  axis=1)
    eid = jnp.minimum(eid, E - 1)
    last_eid = eid[jnp.maximum(n_active - 1, 0)]
    eid = jnp.where(bidx < n_active, eid, last_eid)
    return p0, p1, eid, n_active.reshape(1)


_TOKS_PER_W = N // NW


def _dispatch_sc(x_flat, p0_2d, p1_2d):
    mesh = plsc.VectorSubcoreMesh(core_axis_name="c", subcore_axis_name="s")

    @functools.partial(
        pl.kernel,
        out_type=jax.ShapeDtypeStruct((NPAD, C), jnp.float32),
        mesh=mesh,
        scratch_types=[
            pltpu.VMEM((_TOKS_PER_W,), jnp.int32),
            pltpu.VMEM((_TOKS_PER_W,), jnp.int32),
            pltpu.VMEM((_TOKS_PER_W, C), jnp.float32),
            pltpu.SemaphoreType.DMA,
            pltpu.SemaphoreType.DMA,
        ],
    )
    def k(x_hbm, p0_hbm, p1_hbm, out_hbm, p0_v, p1_v, xbuf, sem0, sem1):
        wid = lax.axis_index("s") * SC_NC + lax.axis_index("c")
        base = wid * _TOKS_PER_W
        pltpu.sync_copy(p0_hbm.at[wid], p0_v)
        pltpu.sync_copy(p1_hbm.at[wid], p1_v)
        pltpu.sync_copy(x_hbm.at[pl.ds(base, _TOKS_PER_W), :], xbuf)
        c0 = pltpu.async_copy(xbuf, out_hbm.at[p0_v], sem0)
        c1 = pltpu.async_copy(xbuf, out_hbm.at[p1_v], sem1)
        c0.wait()
        c1.wait()

    return k(x_flat, p0_2d, p1_2d)


def _gelu_exact(v):
    return 0.5 * v * (1.0 + lax.erf(v * (2.0 ** -0.5)))


def _grouped_body(eid_ref, nact_ref, xb_ref, w1_ref, w2_ref, y_ref):
    b = pl.program_id(0)

    @pl.when(b < nact_ref[0])
    def _():
        xb = xb_ref[...].astype(jnp.bfloat16)
        w1 = w1_ref[0].astype(jnp.bfloat16)
        h = jnp.dot(xb, w1, preferred_element_type=jnp.float32)
        h = _gelu_exact(h).astype(jnp.bfloat16)
        w2 = w2_ref[0].astype(jnp.bfloat16)
        y_ref[...] = jnp.dot(h, w2, preferred_element_type=jnp.float32)


def _grouped_matmul(x_pad, W1, W2, eid, n_active):
    grid_spec = pltpu.PrefetchScalarGridSpec(
        num_scalar_prefetch=2,
        grid=(NBLK,),
        in_specs=[
            pl.BlockSpec((BM, C),
                         lambda b, eid, na: (jnp.minimum(b, na[0] - 1), 0)),
            pl.BlockSpec((1, C, HR), lambda b, eid, na: (eid[b], 0, 0)),
            pl.BlockSpec((1, HR, C), lambda b, eid, na: (eid[b], 0, 0)),
        ],
        out_specs=pl.BlockSpec(
            (BM, C), lambda b, eid, na: (jnp.minimum(b, na[0] - 1), 0)),
    )
    return pl.pallas_call(
        _grouped_body,
        grid_spec=grid_spec,
        out_shape=jax.ShapeDtypeStruct((NPAD, C), jnp.float32),
    )(eid, n_active, x_pad, W1, W2)


def _shared_body(x_ref, w1_ref, w3_ref, w2_ref, o_ref):
    xb = x_ref[...].astype(jnp.bfloat16)
    a = jnp.dot(xb, w1_ref[...].astype(jnp.bfloat16),
                preferred_element_type=jnp.float32)
    g = jnp.dot(xb, w3_ref[...].astype(jnp.bfloat16),
                preferred_element_type=jnp.float32)
    h = ((a * jax.nn.sigmoid(a)) * g).astype(jnp.bfloat16)
    o_ref[...] = jnp.dot(h, w2_ref[...].astype(jnp.bfloat16),
                         preferred_element_type=jnp.float32)


def _shared(x_flat, Ws1, Ws3, Ws2):
    nb = N // 256
    return pl.pallas_call(
        _shared_body,
        grid=(nb,),
        in_specs=[
            pl.BlockSpec((256, C), lambda b: (b, 0)),
            pl.BlockSpec((C, HS), lambda b: (0, 0)),
            pl.BlockSpec((C, HS), lambda b: (0, 0)),
            pl.BlockSpec((HS, C), lambda b: (0, 0)),
        ],
        out_specs=pl.BlockSpec((256, C), lambda b: (b, 0)),
        out_shape=jax.ShapeDtypeStruct((N, C), jnp.float32),
    )(x_flat, Ws1, Ws3, Ws2)


_TOK_PER_W = N // NW
_CCHUNK = 8
_CSLOTS = 4


def _combine_sc(shared, y, p0, p1, g0r, g1r):
    mesh = plsc.VectorSubcoreMesh(core_axis_name="c", subcore_axis_name="s")
    inv3 = jnp.float32(1.0 / (1 + TOP_K))

    nchunks = _TOK_PER_W // _CCHUNK

    @functools.partial(
        pl.kernel,
        out_type=jax.ShapeDtypeStruct((N, C), jnp.float32),
        mesh=mesh,
        scratch_types=[
            pltpu.VMEM((_TOK_PER_W,), jnp.int32),
            pltpu.VMEM((_TOK_PER_W,), jnp.int32),
            pltpu.VMEM((_TOK_PER_W, 16), jnp.float32),
            pltpu.VMEM((_TOK_PER_W, 16), jnp.float32),
            pltpu.VMEM((_CSLOTS, _CCHUNK, C), jnp.float32),
            pltpu.VMEM((_CSLOTS, _CCHUNK, C), jnp.float32),
            pltpu.VMEM((_CSLOTS, _CCHUNK, C), jnp.float32),
            pltpu.VMEM((_CCHUNK, C), jnp.float32),
            pltpu.SemaphoreType.DMA,
            pltpu.SemaphoreType.DMA,
            pltpu.SemaphoreType.DMA,
            pltpu.SemaphoreType.DMA,
            pltpu.SemaphoreType.DMA,
        ],
    )
    def k(sh_hbm, y_hbm, p0_hbm, p1_hbm, g0_hbm, g1_hbm, out_hbm,
          p0_v, p1_v, g0_v, g1_v, y0b, y1b, shb, ob,
          sem0, sem1, sem2, sem3, osem):
        wid = lax.axis_index("s") * SC_NC + lax.axis_index("c")
        base = wid * _TOK_PER_W
        pltpu.sync_copy(p0_hbm.at[pl.ds(base, _TOK_PER_W)], p0_v)
        pltpu.sync_copy(p1_hbm.at[pl.ds(base, _TOK_PER_W)], p1_v)
        pltpu.sync_copy(g0_hbm.at[pl.ds(base, _TOK_PER_W), :], g0_v)
        pltpu.sync_copy(g1_hbm.at[pl.ds(base, _TOK_PER_W), :], g1_v)
        sems = (sem0, sem1, sem2, sem3)

        def gather(cc):
            sl = cc % _CSLOTS
            c0 = pltpu.async_copy(
                y_hbm.at[p0_v.at[pl.ds(cc * _CCHUNK, _CCHUNK)]], y0b.at[sl],
                sems[sl])
            c1 = pltpu.async_copy(
                y_hbm.at[p1_v.at[pl.ds(cc * _CCHUNK, _CCHUNK)]], y1b.at[sl],
                sems[sl])
            c2 = pltpu.async_copy(
                sh_hbm.at[pl.ds(base + cc * _CCHUNK, _CCHUNK), :], shb.at[sl],
                sems[sl])
            return c0, c1, c2

        pend = [gather(cc) for cc in range(_CSLOTS - 1)]
        owait = None
        for cc in range(nchunks):
            sl = cc % _CSLOTS
            if cc + _CSLOTS - 1 < nchunks:
                pend.append(gather(cc + _CSLOTS - 1))
            for c in pend.pop(0):
                c.wait()
            if owait is not None:
                owait.wait()

            def tok(t, _):
                g0vec = g0_v[cc * _CCHUNK + t, :]
                g1vec = g1_v[cc * _CCHUNK + t, :]
                for ch in range(C // 16):
                    s2 = pl.ds(ch * 16, 16)
                    ob[t, s2] = (shb[sl, t, s2]
                                 + g0vec * y0b[sl, t, s2]
                                 + g1vec * y1b[sl, t, s2]) * inv3
                return 0

            lax.fori_loop(0, _CCHUNK, tok, 0)
            owait = pltpu.async_copy(
                ob, out_hbm.at[pl.ds(base + cc * _CCHUNK, _CCHUNK), :], osem)
        owait.wait()

    return k(shared, y, p0, p1, g0r, g1r)


def kernel(x, t_emb, Ws1, Ws3, Ws2, W1, W2, Wr, router_bias):
    B, T, Cc = x.shape
    x_flat = x.reshape(-1, Cc)
    idx2, g0r, g1r = _router(x_flat, t_emb, Wr, router_bias)
    p0, p1, eid, n_active = _dispatch_plan(idx2)
    x_pad = _dispatch_sc(x_flat, p0.reshape(NW, _TOKS_PER_W),
                         p1.reshape(NW, _TOKS_PER_W))
    y = _grouped_matmul(x_pad, W1, W2, eid, n_active)
    sh = _shared(x_flat, Ws1, Ws3, Ws2)
    out = _combine_sc(sh, y, p0, p1, g0r, g1r)
    return out.reshape(B, T, Cc)


# speedup vs baseline: 1.1779x; 1.0502x over previous
import functools

import jax
import jax.numpy as jnp
from jax import lax
from jax.experimental import pallas as pl
from jax.experimental.pallas import tpu as pltpu
from jax.experimental.pallas import tpu_sc as plsc

E = 8
TOP_K = 2
N = 2048
C = 1024
HS = 2048
HR = 1024
BM = 256
NBLK = N * TOP_K // BM + E
NPAD = NBLK * BM

SC_NC = 2
SC_NS = 16
NW = SC_NC * SC_NS


def _router_body(x_ref, wrx_ref, wrt_ref, temb_ref, bias_ref,
                 pos_ref, g0r_ref, g1r_ref, eid_ref, na_ref):
    x = x_ref[...]
    logits = jnp.dot(x, wrx_ref[...], preferred_element_type=jnp.float32)
    tlog = jnp.dot(temb_ref[...], wrt_ref[...],
                   preferred_element_type=jnp.float32)
    logits = logits + tlog
    s = jax.nn.sigmoid(logits)
    cols = lax.broadcasted_iota(jnp.int32, s.shape, 1)
    valid = cols < E
    sel = s + bias_ref[...]
    m1 = jnp.max(sel, axis=1, keepdims=True)
    i1 = jnp.min(jnp.where((sel == m1) & valid, cols, 9999), axis=1,
                 keepdims=True)
    oh0 = cols == i1
    s1 = jnp.sum(jnp.where(oh0, s, 0.0), axis=1, keepdims=True)
    sel2 = jnp.where(oh0, -jnp.float32(3e38), sel)
    m2 = jnp.max(sel2, axis=1, keepdims=True)
    i2 = jnp.min(jnp.where((sel2 == m2) & valid, cols, 9999), axis=1,
                 keepdims=True)
    oh1 = cols == i2
    s2 = jnp.sum(jnp.where(oh1, s, 0.0), axis=1, keepdims=True)
    denom = s1 + s2
    ok = denom > 1e-9
    g0 = jnp.where(ok, s1 / (denom + 1e-9), 0.5)
    g1 = jnp.where(ok, s2 / (denom + 1e-9), 0.5)
    g0r_ref[...] = jnp.broadcast_to(g0, (N, 16))
    g1r_ref[...] = jnp.broadcast_to(g1, (N, 16))

    f0 = oh0.astype(jnp.float32)
    f1 = oh1.astype(jnp.float32)
    cnt = f0 + f1
    incl = cnt
    sh = 1
    while sh < N:
        incl = incl + jnp.concatenate(
            [jnp.zeros((sh, 128), jnp.float32), incl[:-sh]], axis=0)
        sh *= 2
    excl = incl - cnt
    lane_ok = cols[:1] < E
    counts = jnp.where(lane_ok, incl[-1:], 0.0)
    padded = jnp.float32(BM) * jnp.ceil(counts * (1.0 / BM))
    padded = jnp.where(lane_ok, padded, 0.0)
    ltri = (lax.broadcasted_iota(jnp.int32, (128, 128), 0)
            <= lax.broadcasted_iota(jnp.int32, (128, 128), 1))
    pends = jnp.dot(padded, ltri.astype(jnp.float32),
                    preferred_element_type=jnp.float32)
    pstarts = pends - padded
    n_act = jnp.sum(jnp.where(cols[:1] == E - 1, pends, 0.0)) * (1.0 / BM)
    p0 = jnp.sum((pstarts + excl) * f0, axis=1, keepdims=True)
    p1 = jnp.sum((pstarts + excl) * f1, axis=1, keepdims=True)
    pos_ref[...] = jnp.concatenate([p0, p1], axis=1).astype(jnp.int32)
    q = jnp.dot(jnp.ones((128, 1), jnp.float32), pends,
                preferred_element_type=jnp.float32)
    q = jnp.where(cols[:1] < E, q, 3e8)
    brow = lax.broadcasted_iota(jnp.int32, (128, 1), 0).astype(jnp.float32)
    m = (brow * jnp.float32(BM) >= q).astype(jnp.float32)
    eid_f = jnp.minimum(jnp.sum(m, axis=1, keepdims=True), jnp.float32(E - 1))
    last_eid = jnp.sum(jnp.where(brow == n_act - 1.0, eid_f, 0.0))
    eid_f = jnp.where(brow < n_act, eid_f, last_eid)
    eid_ref[...] = eid_f.astype(jnp.int32)
    na_ref[...] = jnp.full((1, 128), n_act, jnp.float32).astype(jnp.int32)


def _router(x_flat, t_emb, Wr, router_bias):
    wrx = jnp.pad(Wr[:C], ((0, 0), (0, 128 - E)))
    wrt = jnp.pad(Wr[C:], ((0, 0), (0, 128 - E)))
    bias = jnp.pad(router_bias, (0, 128 - E), constant_values=-1e30)
    bias = bias.reshape(1, 128)
    return pl.pallas_call(
        _router_body,
        out_shape=(
            jax.ShapeDtypeStruct((N, TOP_K), jnp.int32),
            jax.ShapeDtypeStruct((N, 16), jnp.float32),
            jax.ShapeDtypeStruct((N, 16), jnp.float32),
            jax.ShapeDtypeStruct((128, 1), jnp.int32),
            jax.ShapeDtypeStruct((1, 128), jnp.int32),
        ),
    )(x_flat, wrx, wrt, t_emb, bias)


def _dispatch_plan(idx2):
    ar8 = jnp.arange(E, dtype=jnp.int32)
    oh0 = (idx2[:, :1] == ar8[None, :]).astype(jnp.int32)
    oh1 = (idx2[:, 1:] == ar8[None, :]).astype(jnp.int32)
    cnt = oh0 + oh1
    incl = jnp.cumsum(cnt, axis=0)
    excl = incl - cnt
    counts = incl[-1]
    nblocks = (counts + BM - 1) // BM
    padded = nblocks * BM
    pends = jnp.cumsum(padded)
    pstarts = pends - padded
    n_active = pends[-1] // BM
    p0 = jnp.sum((pstarts[None, :] + excl) * oh0, axis=1)
    p1 = jnp.sum((pstarts[None, :] + excl) * oh1, axis=1)
    bidx = jnp.arange(NBLK, dtype=jnp.int32)
    eid = jnp.sum((bidx[:, None] * BM >= pends[None, :]).astype(jnp.int32),
                  axis=1)
    eid = jnp.minimum(eid, E - 1)
    last_eid = eid[jnp.maximum(n_active - 1, 0)]
    eid = jnp.where(bidx < n_active, eid, last_eid)
    return p0, p1, eid, n_active.reshape(1)


_TOKS_PER_W = N // NW


def _dispatch_sc(x_flat, p0_2d, p1_2d):
    mesh = plsc.VectorSubcoreMesh(core_axis_name="c", subcore_axis_name="s")

    @functools.partial(
        pl.kernel,
        out_type=jax.ShapeDtypeStruct((NPAD, C), jnp.float32),
        mesh=mesh,
        scratch_types=[
            pltpu.VMEM((_TOKS_PER_W,), jnp.int32),
            pltpu.VMEM((_TOKS_PER_W,), jnp.int32),
            pltpu.VMEM((_TOKS_PER_W, C), jnp.float32),
            pltpu.SemaphoreType.DMA,
            pltpu.SemaphoreType.DMA,
        ],
    )
    def k(x_hbm, p0_hbm, p1_hbm, out_hbm, p0_v, p1_v, xbuf, sem0, sem1):
        wid = lax.axis_index("s") * SC_NC + lax.axis_index("c")
        base = wid * _TOKS_PER_W
        pltpu.sync_copy(p0_hbm.at[wid], p0_v)
        pltpu.sync_copy(p1_hbm.at[wid], p1_v)
        pltpu.sync_copy(x_hbm.at[pl.ds(base, _TOKS_PER_W), :], xbuf)
        c0 = pltpu.async_copy(xbuf, out_hbm.at[p0_v], sem0)
        c1 = pltpu.async_copy(xbuf, out_hbm.at[p1_v], sem1)
        c0.wait()
        c1.wait()

    return k(x_flat, p0_2d, p1_2d)


def _gelu_exact(v):
    return 0.5 * v * (1.0 + lax.erf(v * (2.0 ** -0.5)))


def _grouped_body(eid_ref, nact_ref, xb_ref, w1_ref, w2_ref, y_ref):
    b = pl.program_id(0)

    @pl.when(b < nact_ref[0])
    def _():
        xb = xb_ref[...].astype(jnp.bfloat16)
        w1 = w1_ref[0].astype(jnp.bfloat16)
        h = jnp.dot(xb, w1, preferred_element_type=jnp.float32)
        h = _gelu_exact(h).astype(jnp.bfloat16)
        w2 = w2_ref[0].astype(jnp.bfloat16)
        y_ref[...] = jnp.dot(h, w2, preferred_element_type=jnp.float32)


def _grouped_matmul(x_pad, W1, W2, eid, n_active):
    grid_spec = pltpu.PrefetchScalarGridSpec(
        num_scalar_prefetch=2,
        grid=(NBLK,),
        in_specs=[
            pl.BlockSpec((BM, C),
                         lambda b, eid, na: (jnp.minimum(b, na[0] - 1), 0)),
            pl.BlockSpec((1, C, HR), lambda b, eid, na: (eid[b], 0, 0)),
            pl.BlockSpec((1, HR, C), lambda b, eid, na: (eid[b], 0, 0)),
        ],
        out_specs=pl.BlockSpec(
            (BM, C), lambda b, eid, na: (jnp.minimum(b, na[0] - 1), 0)),
    )
    return pl.pallas_call(
        _grouped_body,
        grid_spec=grid_spec,
        out_shape=jax.ShapeDtypeStruct((NPAD, C), jnp.float32),
    )(eid, n_active, x_pad, W1, W2)


def _shared_body(x_ref, w1_ref, w3_ref, w2_ref, o_ref):
    xb = x_ref[...].astype(jnp.bfloat16)
    a = jnp.dot(xb, w1_ref[...].astype(jnp.bfloat16),
                preferred_element_type=jnp.float32)
    g = jnp.dot(xb, w3_ref[...].astype(jnp.bfloat16),
                preferred_element_type=jnp.float32)
    h = ((a * jax.nn.sigmoid(a)) * g).astype(jnp.bfloat16)
    o_ref[...] = jnp.dot(h, w2_ref[...].astype(jnp.bfloat16),
                         preferred_element_type=jnp.float32)


def _shared(x_flat, Ws1, Ws3, Ws2):
    nb = N // 256
    return pl.pallas_call(
        _shared_body,
        grid=(nb,),
        in_specs=[
            pl.BlockSpec((256, C), lambda b: (b, 0)),
            pl.BlockSpec((C, HS), lambda b: (0, 0)),
            pl.BlockSpec((C, HS), lambda b: (0, 0)),
            pl.BlockSpec((HS, C), lambda b: (0, 0)),
        ],
        out_specs=pl.BlockSpec((256, C), lambda b: (b, 0)),
        out_shape=jax.ShapeDtypeStruct((N, C), jnp.float32),
    )(x_flat, Ws1, Ws3, Ws2)


_TOK_PER_W = N // NW
_CCHUNK = 8
_CSLOTS = 4


def _combine_sc(shared, y, p0, p1, g0r, g1r):
    mesh = plsc.VectorSubcoreMesh(core_axis_name="c", subcore_axis_name="s")
    inv3 = jnp.float32(1.0 / (1 + TOP_K))

    nchunks = _TOK_PER_W // _CCHUNK

    @functools.partial(
        pl.kernel,
        out_type=jax.ShapeDtypeStruct((N, C), jnp.float32),
        mesh=mesh,
        scratch_types=[
            pltpu.VMEM((_TOK_PER_W,), jnp.int32),
            pltpu.VMEM((_TOK_PER_W,), jnp.int32),
            pltpu.VMEM((_TOK_PER_W, 16), jnp.float32),
            pltpu.VMEM((_TOK_PER_W, 16), jnp.float32),
            pltpu.VMEM((_CSLOTS, _CCHUNK, C), jnp.float32),
            pltpu.VMEM((_CSLOTS, _CCHUNK, C), jnp.float32),
            pltpu.VMEM((_CSLOTS, _CCHUNK, C), jnp.float32),
            pltpu.VMEM((_CCHUNK, C), jnp.float32),
            pltpu.SemaphoreType.DMA,
            pltpu.SemaphoreType.DMA,
            pltpu.SemaphoreType.DMA,
            pltpu.SemaphoreType.DMA,
            pltpu.SemaphoreType.DMA,
        ],
    )
    def k(sh_hbm, y_hbm, p0_hbm, p1_hbm, g0_hbm, g1_hbm, out_hbm,
          p0_v, p1_v, g0_v, g1_v, y0b, y1b, shb, ob,
          sem0, sem1, sem2, sem3, osem):
        wid = lax.axis_index("s") * SC_NC + lax.axis_index("c")
        base = wid * _TOK_PER_W
        pltpu.sync_copy(p0_hbm.at[pl.ds(base, _TOK_PER_W)], p0_v)
        pltpu.sync_copy(p1_hbm.at[pl.ds(base, _TOK_PER_W)], p1_v)
        pltpu.sync_copy(g0_hbm.at[pl.ds(base, _TOK_PER_W), :], g0_v)
        pltpu.sync_copy(g1_hbm.at[pl.ds(base, _TOK_PER_W), :], g1_v)
        sems = (sem0, sem1, sem2, sem3)

        def gather(cc):
            sl = cc % _CSLOTS
            c0 = pltpu.async_copy(
                y_hbm.at[p0_v.at[pl.ds(cc * _CCHUNK, _CCHUNK)]], y0b.at[sl],
                sems[sl])
            c1 = pltpu.async_copy(
                y_hbm.at[p1_v.at[pl.ds(cc * _CCHUNK, _CCHUNK)]], y1b.at[sl],
                sems[sl])
            c2 = pltpu.async_copy(
                sh_hbm.at[pl.ds(base + cc * _CCHUNK, _CCHUNK), :], shb.at[sl],
                sems[sl])
            return c0, c1, c2

        pend = [gather(cc) for cc in range(_CSLOTS - 1)]
        owait = None
        for cc in range(nchunks):
            sl = cc % _CSLOTS
            if cc + _CSLOTS - 1 < nchunks:
                pend.append(gather(cc + _CSLOTS - 1))
            for c in pend.pop(0):
                c.wait()
            if owait is not None:
                owait.wait()

            def tok(t, _):
                g0vec = g0_v[cc * _CCHUNK + t, :]
                g1vec = g1_v[cc * _CCHUNK + t, :]
                for ch in range(C // 16):
                    s2 = pl.ds(ch * 16, 16)
                    ob[t, s2] = (shb[sl, t, s2]
                                 + g0vec * y0b[sl, t, s2]
                                 + g1vec * y1b[sl, t, s2]) * inv3
                return 0

            lax.fori_loop(0, _CCHUNK, tok, 0)
            owait = pltpu.async_copy(
                ob, out_hbm.at[pl.ds(base + cc * _CCHUNK, _CCHUNK), :], osem)
        owait.wait()

    return k(shared, y, p0, p1, g0r, g1r)


def kernel(x, t_emb, Ws1, Ws3, Ws2, W1, W2, Wr, router_bias):
    B, T, Cc = x.shape
    x_flat = x.reshape(-1, Cc)
    pos2, g0r, g1r, eid_o, na_o = _router(x_flat, t_emb, Wr, router_bias)
    p0 = pos2[:, 0]
    p1 = pos2[:, 1]
    eid = eid_o[:NBLK, 0]
    n_active = na_o[0, :1]
    x_pad = _dispatch_sc(x_flat, p0.reshape(NW, _TOKS_PER_W),
                         p1.reshape(NW, _TOKS_PER_W))
    y = _grouped_matmul(x_pad, W1, W2, eid, n_active)
    sh = _shared(x_flat, Ws1, Ws3, Ws2)
    out = _combine_sc(sh, y, p0, p1, g0r, g1r)
    return out.reshape(B, T, Cc)


# speedup vs baseline: 1.1783x; 1.0004x over previous
import functools

import jax
import jax.numpy as jnp
from jax import lax
from jax.experimental import pallas as pl
from jax.experimental.pallas import tpu as pltpu
from jax.experimental.pallas import tpu_sc as plsc

E = 8
TOP_K = 2
N = 2048
C = 1024
HS = 2048
HR = 1024
BM = 256
NBLK = N * TOP_K // BM + E
NPAD = NBLK * BM

SC_NC = 2
SC_NS = 16
NW = SC_NC * SC_NS


def _router_body(x_ref, wrx_ref, wrt_ref, temb_ref, bias_ref,
                 pos_ref, g0r_ref, g1r_ref, eid_ref, na_ref):
    x = x_ref[...]
    logits = jnp.dot(x, wrx_ref[...], preferred_element_type=jnp.float32)
    tlog = jnp.dot(temb_ref[...], wrt_ref[...],
                   preferred_element_type=jnp.float32)
    logits = logits + tlog
    s = jax.nn.sigmoid(logits)
    cols = lax.broadcasted_iota(jnp.int32, s.shape, 1)
    valid = cols < E
    sel = s + bias_ref[...]
    m1 = jnp.max(sel, axis=1, keepdims=True)
    i1 = jnp.min(jnp.where((sel == m1) & valid, cols, 9999), axis=1,
                 keepdims=True)
    oh0 = cols == i1
    s1 = jnp.sum(jnp.where(oh0, s, 0.0), axis=1, keepdims=True)
    sel2 = jnp.where(oh0, -jnp.float32(3e38), sel)
    m2 = jnp.max(sel2, axis=1, keepdims=True)
    i2 = jnp.min(jnp.where((sel2 == m2) & valid, cols, 9999), axis=1,
                 keepdims=True)
    oh1 = cols == i2
    s2 = jnp.sum(jnp.where(oh1, s, 0.0), axis=1, keepdims=True)
    denom = s1 + s2
    ok = denom > 1e-9
    g0 = jnp.where(ok, s1 / (denom + 1e-9), 0.5)
    g1 = jnp.where(ok, s2 / (denom + 1e-9), 0.5)
    g0r_ref[...] = jnp.broadcast_to(g0, (N, 16))
    g1r_ref[...] = jnp.broadcast_to(g1, (N, 16))

    f0 = oh0.astype(jnp.float32)
    f1 = oh1.astype(jnp.float32)
    cnt = f0 + f1
    incl = cnt
    sh = 1
    while sh < N:
        incl = incl + jnp.concatenate(
            [jnp.zeros((sh, 128), jnp.float32), incl[:-sh]], axis=0)
        sh *= 2
    excl = incl - cnt
    lane_ok = cols[:1] < E
    counts = jnp.where(lane_ok, incl[-1:], 0.0)
    padded = jnp.float32(BM) * jnp.ceil(counts * (1.0 / BM))
    padded = jnp.where(lane_ok, padded, 0.0)
    ltri = (lax.broadcasted_iota(jnp.int32, (128, 128), 0)
            <= lax.broadcasted_iota(jnp.int32, (128, 128), 1))
    pends = jnp.dot(padded, ltri.astype(jnp.float32),
                    preferred_element_type=jnp.float32)
    pstarts = pends - padded
    n_act = jnp.sum(jnp.where(cols[:1] == E - 1, pends, 0.0)) * (1.0 / BM)
    p0 = jnp.sum((pstarts + excl) * f0, axis=1, keepdims=True)
    p1 = jnp.sum((pstarts + excl) * f1, axis=1, keepdims=True)
    pos_ref[...] = jnp.concatenate([p0, p1], axis=1).astype(jnp.int32)
    q = jnp.dot(jnp.ones((128, 1), jnp.float32), pends,
                preferred_element_type=jnp.float32)
    q = jnp.where(cols[:1] < E, q, 3e8)
    brow = lax.broadcasted_iota(jnp.int32, (128, 1), 0).astype(jnp.float32)
    m = (brow * jnp.float32(BM) >= q).astype(jnp.float32)
    eid_f = jnp.minimum(jnp.sum(m, axis=1, keepdims=True), jnp.float32(E - 1))
    last_eid = jnp.sum(jnp.where(brow == n_act - 1.0, eid_f, 0.0))
    eid_f = jnp.where(brow < n_act, eid_f, last_eid)
    eid_ref[...] = eid_f.astype(jnp.int32)
    na_ref[...] = jnp.full((1, 128), n_act, jnp.float32).astype(jnp.int32)


def _router(x_flat, t_emb, Wr, router_bias):
    wrx = jnp.pad(Wr[:C], ((0, 0), (0, 128 - E)))
    wrt = jnp.pad(Wr[C:], ((0, 0), (0, 128 - E)))
    bias = jnp.pad(router_bias, (0, 128 - E), constant_values=-1e30)
    bias = bias.reshape(1, 128)
    return pl.pallas_call(
        _router_body,
        out_shape=(
            jax.ShapeDtypeStruct((N, TOP_K), jnp.int32),
            jax.ShapeDtypeStruct((N, 16), jnp.float32),
            jax.ShapeDtypeStruct((N, 16), jnp.float32),
            jax.ShapeDtypeStruct((128, 1), jnp.int32),
            jax.ShapeDtypeStruct((1, 128), jnp.int32),
        ),
    )(x_flat, wrx, wrt, t_emb, bias)


_TOKS_PER_W = N // NW


def _dispatch_sc(x_flat, p0_2d, p1_2d):
    mesh = plsc.VectorSubcoreMesh(core_axis_name="c", subcore_axis_name="s")

    @functools.partial(
        pl.kernel,
        out_type=jax.ShapeDtypeStruct((NPAD, C), jnp.float32),
        mesh=mesh,
        scratch_types=[
            pltpu.VMEM((_TOKS_PER_W,), jnp.int32),
            pltpu.VMEM((_TOKS_PER_W,), jnp.int32),
            pltpu.VMEM((_TOKS_PER_W, C), jnp.float32),
            pltpu.SemaphoreType.DMA,
            pltpu.SemaphoreType.DMA,
        ],
    )
    def k(x_hbm, p0_hbm, p1_hbm, out_hbm, p0_v, p1_v, xbuf, sem0, sem1):
        wid = lax.axis_index("s") * SC_NC + lax.axis_index("c")
        base = wid * _TOKS_PER_W
        pltpu.sync_copy(p0_hbm.at[wid], p0_v)
        pltpu.sync_copy(p1_hbm.at[wid], p1_v)
        pltpu.sync_copy(x_hbm.at[pl.ds(base, _TOKS_PER_W), :], xbuf)
        c0 = pltpu.async_copy(xbuf, out_hbm.at[p0_v], sem0)
        c1 = pltpu.async_copy(xbuf, out_hbm.at[p1_v], sem1)
        c0.wait()
        c1.wait()

    return k(x_flat, p0_2d, p1_2d)


def _gelu_exact(v):
    return 0.5 * v * (1.0 + lax.erf(v * (2.0 ** -0.5)))


def _grouped_body(eid_ref, nact_ref, xb_ref, w1_ref, w2_ref, y_ref):
    b = pl.program_id(0)

    @pl.when(b < nact_ref[0])
    def _():
        xb = xb_ref[...].astype(jnp.bfloat16)
        w1 = w1_ref[0].astype(jnp.bfloat16)
        h = jnp.dot(xb, w1, preferred_element_type=jnp.float32)
        h = _gelu_exact(h).astype(jnp.bfloat16)
        w2 = w2_ref[0].astype(jnp.bfloat16)
        y_ref[...] = jnp.dot(h, w2, preferred_element_type=jnp.float32)


def _grouped_matmul(x_pad, W1, W2, eid, n_active):
    grid_spec = pltpu.PrefetchScalarGridSpec(
        num_scalar_prefetch=2,
        grid=(NBLK,),
        in_specs=[
            pl.BlockSpec((BM, C),
                         lambda b, eid, na: (jnp.minimum(b, na[0] - 1), 0)),
            pl.BlockSpec((1, C, HR), lambda b, eid, na: (eid[b], 0, 0)),
            pl.BlockSpec((1, HR, C), lambda b, eid, na: (eid[b], 0, 0)),
        ],
        out_specs=pl.BlockSpec(
            (BM, C), lambda b, eid, na: (jnp.minimum(b, na[0] - 1), 0)),
    )
    return pl.pallas_call(
        _grouped_body,
        grid_spec=grid_spec,
        out_shape=jax.ShapeDtypeStruct((NPAD, C), jnp.float32),
    )(eid, n_active, x_pad, W1, W2)


def _shared_body(x_ref, w1_ref, w3_ref, w2_ref, o_ref):
    xb = x_ref[...].astype(jnp.bfloat16)
    a = jnp.dot(xb, w1_ref[...].astype(jnp.bfloat16),
                preferred_element_type=jnp.float32)
    g = jnp.dot(xb, w3_ref[...].astype(jnp.bfloat16),
                preferred_element_type=jnp.float32)
    h = ((a * jax.nn.sigmoid(a)) * g).astype(jnp.bfloat16)
    o_ref[...] = jnp.dot(h, w2_ref[...].astype(jnp.bfloat16),
                         preferred_element_type=jnp.float32)


def _shared(x_flat, Ws1, Ws3, Ws2):
    nb = N // 256
    return pl.pallas_call(
        _shared_body,
        grid=(nb,),
        in_specs=[
            pl.BlockSpec((256, C), lambda b: (b, 0)),
            pl.BlockSpec((C, HS), lambda b: (0, 0)),
            pl.BlockSpec((C, HS), lambda b: (0, 0)),
            pl.BlockSpec((HS, C), lambda b: (0, 0)),
        ],
        out_specs=pl.BlockSpec((256, C), lambda b: (b, 0)),
        out_shape=jax.ShapeDtypeStruct((N, C), jnp.float32),
    )(x_flat, Ws1, Ws3, Ws2)


_TOK_PER_W = N // NW
_CCHUNK = 8
_CSLOTS = 4


def _combine_sc(shared, y, p0, p1, g0r, g1r):
    mesh = plsc.VectorSubcoreMesh(core_axis_name="c", subcore_axis_name="s")
    inv3 = jnp.float32(1.0 / (1 + TOP_K))

    nchunks = _TOK_PER_W // _CCHUNK

    @functools.partial(
        pl.kernel,
        out_type=jax.ShapeDtypeStruct((N, C), jnp.float32),
        mesh=mesh,
        scratch_types=[
            pltpu.VMEM((_TOK_PER_W,), jnp.int32),
            pltpu.VMEM((_TOK_PER_W,), jnp.int32),
            pltpu.VMEM((_TOK_PER_W, 16), jnp.float32),
            pltpu.VMEM((_TOK_PER_W, 16), jnp.float32),
            pltpu.VMEM((_CSLOTS, _CCHUNK, C), jnp.float32),
            pltpu.VMEM((_CSLOTS, _CCHUNK, C), jnp.float32),
            pltpu.VMEM((_CSLOTS, _CCHUNK, C), jnp.float32),
            pltpu.VMEM((_CCHUNK, C), jnp.float32),
            pltpu.SemaphoreType.DMA,
            pltpu.SemaphoreType.DMA,
            pltpu.SemaphoreType.DMA,
            pltpu.SemaphoreType.DMA,
            pltpu.SemaphoreType.DMA,
        ],
    )
    def k(sh_hbm, y_hbm, p0_hbm, p1_hbm, g0_hbm, g1_hbm, out_hbm,
          p0_v, p1_v, g0_v, g1_v, y0b, y1b, shb, ob,
          sem0, sem1, sem2, sem3, osem):
        wid = lax.axis_index("s") * SC_NC + lax.axis_index("c")
        base = wid * _TOK_PER_W
        pltpu.sync_copy(p0_hbm.at[pl.ds(base, _TOK_PER_W)], p0_v)
        pltpu.sync_copy(p1_hbm.at[pl.ds(base, _TOK_PER_W)], p1_v)
        pltpu.sync_copy(g0_hbm.at[pl.ds(base, _TOK_PER_W), :], g0_v)
        pltpu.sync_copy(g1_hbm.at[pl.ds(base, _TOK_PER_W), :], g1_v)
        sems = (sem0, sem1, sem2, sem3)

        def gather(cc):
            sl = cc % _CSLOTS
            c0 = pltpu.async_copy(
                y_hbm.at[p0_v.at[pl.ds(cc * _CCHUNK, _CCHUNK)]], y0b.at[sl],
                sems[sl])
            c1 = pltpu.async_copy(
                y_hbm.at[p1_v.at[pl.ds(cc * _CCHUNK, _CCHUNK)]], y1b.at[sl],
                sems[sl])
            c2 = pltpu.async_copy(
                sh_hbm.at[pl.ds(base + cc * _CCHUNK, _CCHUNK), :], shb.at[sl],
                sems[sl])
            return c0, c1, c2

        pend = [gather(cc) for cc in range(_CSLOTS - 1)]
        owait = None
        for cc in range(nchunks):
            sl = cc % _CSLOTS
            if cc + _CSLOTS - 1 < nchunks:
                pend.append(gather(cc + _CSLOTS - 1))
            for c in pend.pop(0):
                c.wait()
            if owait is not None:
                owait.wait()

            def tok(t, _):
                g0vec = g0_v[cc * _CCHUNK + t, :]
                g1vec = g1_v[cc * _CCHUNK + t, :]
                for ch in range(C // 16):
                    s2 = pl.ds(ch * 16, 16)
                    ob[t, s2] = (shb[sl, t, s2]
                                 + g0vec * y0b[sl, t, s2]
                                 + g1vec * y1b[sl, t, s2]) * inv3
                return 0

            lax.fori_loop(0, _CCHUNK, tok, 0)
            owait = pltpu.async_copy(
                ob, out_hbm.at[pl.ds(base + cc * _CCHUNK, _CCHUNK), :], osem)
        owait.wait()

    return k(shared, y, p0, p1, g0r, g1r)


def kernel(x, t_emb, Ws1, Ws3, Ws2, W1, W2, Wr, router_bias):
    B, T, Cc = x.shape
    x_flat = x.reshape(-1, Cc)
    pos2, g0r, g1r, eid_o, na_o = _router(x_flat, t_emb, Wr, router_bias)
    p0 = pos2[:, 0]
    p1 = pos2[:, 1]
    eid = eid_o[:NBLK, 0]
    n_active = na_o[0, :1]
    x_pad = _dispatch_sc(x_flat, p0.reshape(NW, _TOKS_PER_W),
                         p1.reshape(NW, _TOKS_PER_W))
    y = _grouped_matmul(x_pad, W1, W2, eid, n_active)
    sh = _shared(x_flat, Ws1, Ws3, Ws2)
    out = _combine_sc(sh, y, p0, p1, g0r, g1r)
    return out.reshape(B, T, Cc)
